# baseline (speedup 1.0000x reference)
import jax
import jax.numpy as jnp
from jax import lax
from jax.experimental import pallas as pl
from jax.experimental.pallas import tpu as pltpu
from jax.experimental.pallas import tpu_sc as plsc

_NUM_KEEP_FIRST = 7
_SLIDING_WINDOW_SIZE = 50
_PSP_ALPHA = 0.6
_ITER_ATTR = 0

_BB = 256
_L = 16


def _colsum_body(trg_ref, src_ref, s1_ref, s2_ref):
    i = pl.program_id(0)
    d = trg_ref[0] - src_ref[0]
    p1 = jnp.sum(d, axis=0, keepdims=True)
    p2 = jnp.sum(jnp.abs(d), axis=0, keepdims=True)
    b1 = jnp.broadcast_to(p1, s1_ref.shape[1:])[None]
    b2 = jnp.broadcast_to(p2, s2_ref.shape[1:])[None]

    @pl.when(i % 2 == 0)
    def _():
        s1_ref[...] = b1
        s2_ref[...] = b2

    @pl.when(i % 2 != 0)
    def _():
        s1_ref[...] = s1_ref[...] + b1
        s2_ref[...] = s2_ref[...] + b2


def _column_sums(trg_t, src_t, rows, cols):
    b = trg_t.shape[1]
    hb = b // 2
    grid = (2 * rows,)
    in_spec = pl.BlockSpec((1, hb, cols), lambda i: (i // 2, i % 2, 0))
    out_spec = pl.BlockSpec((1, 8, cols), lambda i: (i // 2, 0, 0))
    return pl.pallas_call(
        _colsum_body,
        grid=grid,
        in_specs=[in_spec, in_spec],
        out_specs=[out_spec, out_spec],
        out_shape=[
            jax.ShapeDtypeStruct((rows, 8, cols), jnp.float32),
            jax.ShapeDtypeStruct((rows, 8, cols), jnp.float32),
        ],
    )(trg_t, src_t)


def _make_select(keep, narr, k_sel, scale):
    mesh = plsc.VectorSubcoreMesh(core_axis_name="c", subcore_axis_name="s")
    nchunks = keep // _L

    def _scan_hist(hc_v, hs_v, nb, k_rem):
        carry = jnp.int32(0)
        fcarry = jnp.float32(0.0)
        ones = jnp.ones((_L,), jnp.int32)
        zeros = jnp.zeros((_L,), jnp.int32)
        fzeros = jnp.zeros((_L,), jnp.float32)
        below_cnt = zeros
        below_max = zeros
        below_s2 = fzeros
        for j in range(nb // _L):
            h = hc_v[pl.ds(j * _L, _L)]
            h2 = hs_v[pl.ds(j * _L, _L)]
            cs = plsc.cumsum(h) + carry
            fs = plsc.cumsum(h2) + fcarry
            lt = cs < k_rem
            below_cnt = below_cnt + jnp.where(lt, ones, zeros)
            below_max = jnp.maximum(below_max, jnp.where(lt, cs, zeros))
            below_s2 = jnp.maximum(below_s2, jnp.where(lt, fs, fzeros))
            carry = carry + jnp.sum(h)
            fcarry = fcarry + jnp.sum(h2)
        return jnp.sum(below_cnt), jnp.max(below_max), jnp.max(below_s2)

    def body(s1_hbm, s2_hbm, out_hbm, s1_v, s2_v, key_v,
             ck_v, ci_v, cs_v, hc_v, hs_v, out_v):
        cid = lax.axis_index("c")
        sid = lax.axis_index("s")

        @pl.when(jnp.logical_and(cid == 0, sid == 0))
        def _():
            pltpu.sync_copy(s1_hbm, s1_v)
            pltpu.sync_copy(s2_hbm, s2_v)

            ones = jnp.ones((_L,), jnp.int32)
            zeros = jnp.zeros((_L,), jnp.int32)
            fzeros = jnp.zeros((_L,), jnp.float32)
            iota = lax.iota(jnp.int32, _L)

            for j in range(256 // _L):
                hc_v[pl.ds(j * _L, _L)] = zeros
                hs_v[pl.ds(j * _L, _L)] = fzeros

            @pl.loop(0, keep, step=_L)
            def _(c):
                k = plsc.bitcast(jnp.abs(s1_v[pl.ds(c, _L)]), jnp.int32)
                key_v[pl.ds(c, _L)] = k
                bin_ = lax.shift_right_logical(k, 24)
                plsc.addupdate_scatter(hc_v, [bin_], ones)
                plsc.addupdate_scatter(hs_v, [bin_], s2_v[pl.ds(c, _L)])

            b_star, n_below, s2_below = _scan_hist(hc_v, hs_v, 256, k_sel)
            k_rem = jnp.int32(k_sel) - n_below
            acc = s2_below
            prefix = b_star

            def cbody(j, cnt):
                c = j * _L
                k = key_v[pl.ds(c, _L)]
                m = lax.shift_right_logical(k, 24) == prefix
                plsc.store_compressed(ck_v.at[pl.ds(cnt, _L)], k, mask=m)
                plsc.store_compressed(ci_v.at[pl.ds(cnt, _L)], iota + c, mask=m)
                plsc.store_compressed(cs_v.at[pl.ds(cnt, _L)],
                                      s2_v[pl.ds(c, _L)], mask=m)
                return cnt + jnp.sum(m.astype(jnp.int32))

            ncand = lax.fori_loop(0, nchunks, cbody, jnp.int32(0))
            nchd = (ncand + (_L - 1)) // _L

            for r in range(1, 4):
                shift = 24 - 8 * r
                for j in range(256 // _L):
                    hc_v[pl.ds(j * _L, _L)] = zeros
                    hs_v[pl.ds(j * _L, _L)] = fzeros
                pfx = prefix

                def hbody(j, _, _r=r, _shift=shift, _pfx=pfx, _n=ncand):
                    c = j * _L
                    k = ck_v[pl.ds(c, _L)]
                    inb = (iota + c) < _n
                    m = jnp.logical_and(
                        inb, lax.shift_right_logical(k, 32 - 8 * _r) == _pfx)
                    bin_ = lax.shift_right_logical(k, _shift) & 255
                    plsc.addupdate_scatter(hc_v, [bin_], ones, mask=m)
                    plsc.addupdate_scatter(hs_v, [bin_],
                                           cs_v[pl.ds(c, _L)], mask=m)
                    return 0

                lax.fori_loop(0, nchd, hbody, 0)
                b_star, n_below, s2_below = _scan_hist(hc_v, hs_v, 256, k_rem)
                k_rem = k_rem - n_below
                acc = acc + s2_below
                prefix = (prefix << 8) | b_star

            t = prefix
            ipfx = jnp.int32(0)
            for r in range(2):
                shift = 6 - 6 * r
                for j in range(64 // _L):
                    hc_v[pl.ds(j * _L, _L)] = zeros
                    hs_v[pl.ds(j * _L, _L)] = fzeros
                ip = ipfx

                def ibody(j, _, _r=r, _shift=shift, _ip=ip, _n=ncand):
                    c = j * _L
                    k = ck_v[pl.ds(c, _L)]
                    idxv = ci_v[pl.ds(c, _L)]
                    inb = (iota + c) < _n
                    m = jnp.logical_and(inb, k == t)
                    if _r == 1:
                        m = jnp.logical_and(
                            m, lax.shift_right_logical(idxv, 6) == _ip)
                    bin_ = lax.shift_right_logical(idxv, _shift) & 63
                    plsc.addupdate_scatter(hc_v, [bin_], ones, mask=m)
                    plsc.addupdate_scatter(hs_v, [bin_],
                                           cs_v[pl.ds(c, _L)], mask=m)
                    return 0

                lax.fori_loop(0, nchd, ibody, 0)
                b_star, n_below, s2_below = _scan_hist(hc_v, hs_v, 64, k_rem)
                k_rem = k_rem - n_below
                acc = acc + s2_below
                ipfx = (ipfx << 6) | b_star

            bs2 = plsc.load_gather(hs_v, [zeros + b_star])
            total = (acc + jnp.max(bs2)) * scale
            out_v[...] = fzeros + total
            pltpu.sync_copy(out_v, out_hbm)

    return pl.kernel(
        body,
        out_type=jax.ShapeDtypeStruct((_L,), jnp.float32),
        mesh=mesh,
        compiler_params=pltpu.CompilerParams(needs_layout_passes=False),
        scratch_types=[
            pltpu.VMEM((narr,), jnp.float32),
            pltpu.VMEM((narr,), jnp.float32),
            pltpu.VMEM((keep,), jnp.int32),
            pltpu.VMEM((keep + _L,), jnp.int32),
            pltpu.VMEM((keep + _L,), jnp.int32),
            pltpu.VMEM((keep + _L,), jnp.float32),
            pltpu.VMEM((256,), jnp.int32),
            pltpu.VMEM((256,), jnp.float32),
            pltpu.VMEM((_L,), jnp.float32),
        ],
    )


def kernel(trg_latents, src_latents, iters):
    b = trg_latents.shape[0]
    cols = trg_latents.shape[2]
    keep = _NUM_KEEP_FIRST * cols
    k_sel = int(_PSP_ALPHA * keep)
    trg_t = jnp.transpose(trg_latents, (1, 0, 2))
    src_t = jnp.transpose(src_latents, (1, 0, 2))
    s1, s2 = _column_sums(trg_t, src_t, _NUM_KEEP_FIRST, cols)
    scale = 1.0 / (b * keep)
    sel = _make_select(keep, keep, k_sel, scale)
    tot = sel(s1[:, 0, :].reshape(-1), s2[:, 0, :].reshape(-1))
    w = jnp.maximum(0.0, (iters - _SLIDING_WINDOW_SIZE)
                    / (_ITER_ATTR - _SLIDING_WINDOW_SIZE))
    return w * tot[0]


# speedup vs baseline: 1.1364x; 1.1364x over previous
import functools

import jax
import jax.numpy as jnp
from jax import lax
from jax.experimental import pallas as pl
from jax.experimental.pallas import tpu as pltpu
from jax.experimental.pallas import tpu_sc as plsc

_NUM_KEEP_FIRST = 7
_SLIDING_WINDOW_SIZE = 50
_PSP_ALPHA = 0.6
_ITER_ATTR = 0

_BB = 256
_L = 16


def _colsum_body(trg_ref, src_ref, s1_ref, s2_ref):
    i = pl.program_id(0)
    d = trg_ref[...] - src_ref[...]
    p1 = jnp.sum(d, axis=1)
    p2 = jnp.sum(jnp.abs(d), axis=1)

    @pl.when(i == 0)
    def _():
        s1_ref[...] = p1
        s2_ref[...] = p2

    @pl.when(i != 0)
    def _():
        s1_ref[...] = s1_ref[...] + p1
        s2_ref[...] = s2_ref[...] + p2


def _column_sums(trg_t, src_t, rows, cols):
    b = trg_t.shape[1]
    grid = (b // _BB,)
    return pl.pallas_call(
        _colsum_body,
        grid=grid,
        in_specs=[
            pl.BlockSpec((rows, _BB, cols), lambda i: (0, i, 0)),
            pl.BlockSpec((rows, _BB, cols), lambda i: (0, i, 0)),
        ],
        out_specs=[
            pl.BlockSpec((rows, cols), lambda i: (0, 0)),
            pl.BlockSpec((rows, cols), lambda i: (0, 0)),
        ],
        out_shape=[
            jax.ShapeDtypeStruct((rows, cols), jnp.float32),
            jax.ShapeDtypeStruct((rows, cols), jnp.float32),
        ],
    )(trg_t, src_t)


def _make_select(keep, narr, k_sel, scale):
    mesh = plsc.VectorSubcoreMesh(core_axis_name="c", subcore_axis_name="s",
                                  num_cores=1)
    nchunks = keep // _L

    def _scan_hist(hist_v, nb, k_rem):
        carry = jnp.int32(0)
        ones = jnp.ones((_L,), jnp.int32)
        zeros = jnp.zeros((_L,), jnp.int32)
        below_cnt = zeros
        below_max = zeros
        for j in range(nb // _L):
            h = hist_v[pl.ds(j * _L, _L)]
            cs = plsc.cumsum(h) + carry
            lt = cs < k_rem
            below_cnt = below_cnt + jnp.where(lt, ones, zeros)
            below_max = jnp.maximum(below_max, jnp.where(lt, cs, zeros))
            carry = carry + jnp.sum(h)
        return jnp.sum(below_cnt), jnp.max(below_max)

    def body(s1_hbm, s2_hbm, out_hbm, s1_v, s2_v, key_v, hist_v, out_v):
        cid = lax.axis_index("c")
        sid = lax.axis_index("s")

        @pl.when(jnp.logical_and(cid == 0, sid == 0))
        def _():
            pltpu.sync_copy(s1_hbm, s1_v)
            pltpu.sync_copy(s2_hbm, s2_v)

            @pl.loop(0, keep, step=_L)
            def _(c):
                key_v[pl.ds(c, _L)] = plsc.bitcast(
                    jnp.abs(s1_v[pl.ds(c, _L)]), jnp.int32)

            ones = jnp.ones((_L,), jnp.int32)
            zeros = jnp.zeros((_L,), jnp.int32)
            iota = lax.iota(jnp.int32, _L)

            k_rem = jnp.int32(k_sel)
            prefix = jnp.int32(0)
            for r in range(4):
                shift = 24 - 8 * r
                for j in range(256 // _L):
                    hist_v[pl.ds(j * _L, _L)] = zeros
                pfx = prefix

                @pl.loop(0, keep, step=_L)
                def _(c, _r=r, _shift=shift, _pfx=pfx):
                    k = key_v[pl.ds(c, _L)]
                    bin_ = lax.shift_right_logical(k, _shift) & 255
                    if _r == 0:
                        plsc.addupdate_scatter(hist_v, [bin_], ones)
                    else:
                        m = lax.shift_right_logical(k, 32 - 8 * _r) == _pfx
                        plsc.addupdate_scatter(hist_v, [bin_], ones, mask=m)

                b_star, n_below = _scan_hist(hist_v, 256, k_rem)
                k_rem = k_rem - n_below
                prefix = (prefix << 8) | b_star

            t = prefix
            ipfx = jnp.int32(0)
            for r in range(2):
                shift = 6 - 6 * r
                for j in range(64 // _L):
                    hist_v[pl.ds(j * _L, _L)] = zeros
                ip = ipfx

                @pl.loop(0, keep, step=_L)
                def _(c, _r=r, _shift=shift, _ip=ip):
                    k = key_v[pl.ds(c, _L)]
                    idxv = iota + c
                    if _r == 0:
                        m = k == t
                    else:
                        m = jnp.logical_and(
                            k == t, lax.shift_right_logical(idxv, 6) == _ip)
                    bin_ = lax.shift_right_logical(idxv, _shift) & 63
                    plsc.addupdate_scatter(hist_v, [bin_], ones, mask=m)

                b_star, n_below = _scan_hist(hist_v, 64, k_rem)
                k_rem = k_rem - n_below
                ipfx = (ipfx << 6) | b_star

            idx_t = ipfx

            def fbody(i, acc):
                c = i * _L
                k = key_v[pl.ds(c, _L)]
                idxv = iota + c
                sel = jnp.logical_or(
                    k < t, jnp.logical_and(k == t, idxv <= idx_t))
                return acc + jnp.where(sel, s2_v[pl.ds(c, _L)],
                                       jnp.zeros((_L,), jnp.float32))

            acc = lax.fori_loop(0, nchunks, fbody,
                                jnp.zeros((_L,), jnp.float32))
            total = jnp.sum(acc) * scale
            out_v[...] = jnp.zeros((_L,), jnp.float32) + total
            pltpu.sync_copy(out_v, out_hbm)

    return pl.kernel(
        body,
        out_type=jax.ShapeDtypeStruct((_L,), jnp.float32),
        mesh=mesh,
        compiler_params=pltpu.CompilerParams(needs_layout_passes=False),
        scratch_types=[
            pltpu.VMEM((narr,), jnp.float32),
            pltpu.VMEM((narr,), jnp.float32),
            pltpu.VMEM((keep,), jnp.int32),
            pltpu.VMEM((256,), jnp.int32),
            pltpu.VMEM((_L,), jnp.float32),
        ],
    )


def kernel(trg_latents, src_latents, iters):
    b = trg_latents.shape[0]
    cols = trg_latents.shape[2]
    keep = _NUM_KEEP_FIRST * cols
    k_sel = int(_PSP_ALPHA * keep)
    trg_t = jnp.transpose(trg_latents, (1, 0, 2))
    src_t = jnp.transpose(src_latents, (1, 0, 2))
    s1, s2 = _column_sums(trg_t, src_t, _NUM_KEEP_FIRST, cols)
    scale = 1.0 / (b * keep)
    sel = _make_select(keep, keep, k_sel, scale)
    tot = sel(s1.reshape(-1), s2.reshape(-1))
    w = jnp.maximum(0.0, (iters - _SLIDING_WINDOW_SIZE)
                    / (_ITER_ATTR - _SLIDING_WINDOW_SIZE))
    return w * tot[0]
